# baseline (speedup 1.0000x reference)
import functools

import jax
import jax.numpy as jnp
from jax import lax
from jax.experimental import pallas as pl
from jax.experimental.pallas import tpu as pltpu
from jax.experimental.pallas import tpu_sc as plsc

N = 10000
E = 320000
D = 128

NC = 2
NS = 16
NW = NC * NS
C = 128
CH_PER_TILE = 80
E_PAD = NW * CH_PER_TILE * C
NACC = 10240
DUMMY = N
ACC_PER_TILE = NACC // NS
ROWS_PER_TILE = 624
TAIL_R0 = ROWS_PER_TILE * NS
TAIL_ROWS = N - TAIL_R0


def _copy_rows(copy_fn, s):
    copy_fn(s * ROWS_PER_TILE, ROWS_PER_TILE)

    @pl.when(s == NS - 1)
    def _():
        copy_fn(TAIL_R0, TAIL_ROWS)

_mesh = plsc.VectorSubcoreMesh(core_axis_name="c", subcore_axis_name="s")


def _init_accum(zeros_hbm, accum, s):
    r0 = s * ACC_PER_TILE
    pltpu.sync_copy(zeros_hbm.at[pl.ds(r0, ACC_PER_TILE)],
                    accum.at[pl.ds(r0, ACC_PER_TILE)])


def _writeback(accum, out_hbm, c, s):
    _copy_rows(lambda r0, nr: pltpu.sync_copy(
        accum.at[pl.ds(r0, nr)], out_hbm.at[c, pl.ds(r0, nr)]), s)


@functools.partial(
    pl.kernel,
    mesh=_mesh,
    out_type=jax.ShapeDtypeStruct((NC, N, D), jnp.float32),
    scratch_types=[
        pltpu.VMEM((CH_PER_TILE, C), jnp.int32),
        pltpu.VMEM((C, D), jnp.float32),
        pltpu.VMEM_SHARED((NACC, D), jnp.float32),
    ],
)
def _deg_kernel(dst_hbm, ones_hbm, zeros_hbm, out_hbm, dst_all, ones_v, accum):
    c = lax.axis_index("c")
    s = lax.axis_index("s")
    wid = s * NC + c
    _init_accum(zeros_hbm, accum, s)
    pltpu.sync_copy(dst_hbm.at[pl.ds(wid * CH_PER_TILE, CH_PER_TILE)], dst_all)
    pltpu.sync_copy(ones_hbm, ones_v)
    plsc.subcore_barrier()

    def body(j, carry):
        pltpu.sync_copy(ones_v, accum.at[dst_all.at[j]], add=True)
        return carry

    lax.fori_loop(0, CH_PER_TILE, body, 0)
    plsc.subcore_barrier()
    _writeback(accum, out_hbm, c, s)


@functools.partial(
    pl.kernel,
    mesh=_mesh,
    out_type=jax.ShapeDtypeStruct((NC, N, D), jnp.float32),
    scratch_types=[
        pltpu.VMEM((C,), jnp.int32),
        pltpu.VMEM((C,), jnp.int32),
        pltpu.VMEM((C,), jnp.int32),
        pltpu.VMEM((C, D), jnp.float32),
        pltpu.VMEM((C, D), jnp.float32),
        pltpu.VMEM_SHARED((NACC, D), jnp.float32),
        pltpu.SemaphoreType.DMA,
        pltpu.SemaphoreType.DMA,
    ],
)
def _spmm_kernel(g_hbm, src_hbm, dst_hbm, zeros_hbm, out_hbm,
                 dbuf, sbuf0, sbuf1, rows_a, rows_b, accum,
                 sa, sb):
    c = lax.axis_index("c")
    s = lax.axis_index("s")
    wid = c * NS + s
    base = wid * CH_PER_TILE
    _init_accum(zeros_hbm, accum, s)
    plsc.subcore_barrier()

    def src_slice(j):
        return src_hbm.at[pl.ds((base + j) * C, C)]

    def dst_slice(j):
        return dst_hbm.at[pl.ds((base + j) * C, C)]

    def body(j, carry):
        pltpu.sync_copy(src_slice(j), sbuf0)
        pltpu.sync_copy(dst_slice(j), dbuf)
        pltpu.async_copy(g_hbm.at[sbuf0], rows_a, sa).wait()
        pltpu.sync_copy(rows_a, accum.at[dbuf], add=True)
        return carry

    lax.fori_loop(0, CH_PER_TILE, body, 0)
    plsc.subcore_barrier()
    _writeback(accum, out_hbm, c, s)


NB = 1000
GRID = N // NB


def _first_body(p_ref, x_ref, w_ref, g_ref, u_ref):
    p = p_ref[...]
    deg = p[0, :, :1] + p[1, :, :1] + 1.0
    u = lax.rsqrt(deg)
    u_ref[...] = jnp.broadcast_to(u, (NB, 16))
    h = jnp.dot(x_ref[...], w_ref[...], preferred_element_type=jnp.float32,
                precision=lax.Precision.HIGHEST)
    g_ref[...] = h * u


def _mid_body(s_ref, g_ref, u_ref, b_ref, w_ref, o_ref):
    sv = s_ref[...]
    u = u_ref[...][:, :1]
    t = (sv[0] + sv[1] + g_ref[...]) * u + b_ref[...]
    z = jnp.maximum(t, 0.0)
    o_ref[...] = jnp.dot(z, w_ref[...], preferred_element_type=jnp.float32,
                         precision=lax.Precision.HIGHEST) * u


def _last_body(s_ref, g_ref, u_ref, b_ref, o_ref):
    sv = s_ref[...]
    u = u_ref[...][:, :1]
    o_ref[...] = (sv[0] + sv[1] + g_ref[...]) * u + b_ref[...]


_spec_p = pl.BlockSpec((2, NB, D), lambda i: (0, i, 0))
_spec_x = pl.BlockSpec((NB, D), lambda i: (i, 0))
_spec_w = pl.BlockSpec((D, D), lambda i: (0, 0))
_spec_s = pl.BlockSpec((2, NB, D), lambda i: (0, i, 0))
_spec_u = pl.BlockSpec((NB, 16), lambda i: (i, 0))
_spec_b = pl.BlockSpec((1, D), lambda i: (0, 0))

_first_tc = pl.pallas_call(
    _first_body,
    grid=(GRID,),
    in_specs=[_spec_p, _spec_x, _spec_w],
    out_specs=[_spec_x, _spec_u],
    out_shape=[jax.ShapeDtypeStruct((N, D), jnp.float32),
               jax.ShapeDtypeStruct((N, 16), jnp.float32)],
)

_mid_tc = pl.pallas_call(
    _mid_body,
    grid=(GRID,),
    in_specs=[_spec_s, _spec_x, _spec_u, _spec_b, _spec_w],
    out_specs=_spec_x,
    out_shape=jax.ShapeDtypeStruct((N, D), jnp.float32),
)

_last_tc = pl.pallas_call(
    _last_body,
    grid=(GRID,),
    in_specs=[_spec_s, _spec_x, _spec_u, _spec_b],
    out_specs=_spec_x,
    out_shape=jax.ShapeDtypeStruct((N, D), jnp.float32),
)


@jax.jit
def kernel(x, adj_t, W1, b1, W2, b2, W3, b3):
    adj = adj_t.astype(jnp.int32)
    src = jnp.concatenate([adj[0], jnp.zeros((E_PAD - E,), jnp.int32)])
    pad_dst = DUMMY + jnp.arange(E_PAD - E, dtype=jnp.int32) % (NACC - N)
    dst = jnp.concatenate([adj[1], pad_dst])
    dst2d = dst.reshape(-1, C)
    onesCD = jnp.ones((C, D), jnp.float32)
    zerosAD = jnp.zeros((NACC, D), jnp.float32)

    p = _deg_kernel(dst2d, onesCD, zerosAD)
    g1, u16 = _first_tc(p, x, W1)
    s1 = _spmm_kernel(g1, src, dst, zerosAD)
    g2 = _mid_tc(s1, g1, u16, b1.reshape(1, D), W2)
    s2 = _spmm_kernel(g2, src, dst, zerosAD)
    g3 = _mid_tc(s2, g2, u16, b2.reshape(1, D), W3)
    s3 = _spmm_kernel(g3, src, dst, zerosAD)
    out = _last_tc(s3, g3, u16, b3.reshape(1, D))
    return out


# speedup vs baseline: 1.0645x; 1.0645x over previous
import functools

import jax
import jax.numpy as jnp
from jax import lax
from jax.experimental import pallas as pl
from jax.experimental.pallas import tpu as pltpu
from jax.experimental.pallas import tpu_sc as plsc

N = 10000
E = 320000
D = 128

NC = 2
NS = 16
NW = NC * NS
C = 128
CH_PER_TILE = 80
E_PAD = NW * CH_PER_TILE * C
NACC = 10240
DUMMY = N
ACC_PER_TILE = NACC // NS
ROWS_PER_TILE = 624
TAIL_R0 = ROWS_PER_TILE * NS
TAIL_ROWS = N - TAIL_R0


def _copy_rows(copy_fn, s):
    copy_fn(s * ROWS_PER_TILE, ROWS_PER_TILE)

    @pl.when(s == NS - 1)
    def _():
        copy_fn(TAIL_R0, TAIL_ROWS)

_mesh = plsc.VectorSubcoreMesh(core_axis_name="c", subcore_axis_name="s")


def _init_accum(zeros_hbm, accum, s):
    r0 = s * ACC_PER_TILE
    pltpu.sync_copy(zeros_hbm.at[pl.ds(r0, ACC_PER_TILE)],
                    accum.at[pl.ds(r0, ACC_PER_TILE)])


def _writeback(accum, out_hbm, c, s):
    _copy_rows(lambda r0, nr: pltpu.sync_copy(
        accum.at[pl.ds(r0, nr)], out_hbm.at[c, pl.ds(r0, nr)]), s)


@functools.partial(
    pl.kernel,
    mesh=_mesh,
    out_type=jax.ShapeDtypeStruct((NC, N, D), jnp.float32),
    scratch_types=[
        pltpu.VMEM((CH_PER_TILE, C), jnp.int32),
        pltpu.VMEM((C, D), jnp.float32),
        pltpu.VMEM_SHARED((NACC, D), jnp.float32),
    ],
)
def _deg_kernel(dst_hbm, ones_hbm, zeros_hbm, out_hbm, dst_all, ones_v, accum):
    c = lax.axis_index("c")
    s = lax.axis_index("s")
    wid = s * NC + c
    _init_accum(zeros_hbm, accum, s)
    pltpu.sync_copy(dst_hbm.at[pl.ds(wid * CH_PER_TILE, CH_PER_TILE)], dst_all)
    pltpu.sync_copy(ones_hbm, ones_v)
    plsc.subcore_barrier()

    def body(j, carry):
        pltpu.sync_copy(ones_v, accum.at[dst_all.at[j]], add=True)
        return carry

    lax.fori_loop(0, CH_PER_TILE, body, 0)
    plsc.subcore_barrier()
    _writeback(accum, out_hbm, c, s)


@functools.partial(
    pl.kernel,
    mesh=_mesh,
    out_type=jax.ShapeDtypeStruct((NC, N, D), jnp.float32),
    scratch_types=[
        pltpu.VMEM((C,), jnp.int32),
        pltpu.VMEM((C,), jnp.int32),
        pltpu.VMEM((C,), jnp.int32),
        pltpu.VMEM((C, D), jnp.float32),
        pltpu.VMEM((C, D), jnp.float32),
        pltpu.VMEM_SHARED((NACC, D), jnp.float32),
        pltpu.SemaphoreType.DMA,
        pltpu.SemaphoreType.DMA,
    ],
)
def _spmm_kernel(g_hbm, src_hbm, dst_hbm, zeros_hbm, out_hbm,
                 dbuf, sbuf0, sbuf1, rows_a, rows_b, accum,
                 sa, sb):
    c = lax.axis_index("c")
    s = lax.axis_index("s")
    wid = s * NC + c
    _init_accum(zeros_hbm, accum, s)
    plsc.subcore_barrier()

    def src_slice(j):
        return src_hbm.at[pl.ds((j * NW + wid) * C, C)]

    def dst_slice(j):
        return dst_hbm.at[pl.ds((j * NW + wid) * C, C)]

    def body(j, carry):
        pltpu.sync_copy(src_slice(j), sbuf0)
        pltpu.sync_copy(dst_slice(j), dbuf)
        pltpu.async_copy(g_hbm.at[sbuf0], rows_a, sa).wait()
        pltpu.sync_copy(rows_a, accum.at[dbuf], add=True)
        return carry

    lax.fori_loop(0, CH_PER_TILE, body, 0)
    plsc.subcore_barrier()
    _writeback(accum, out_hbm, c, s)


NB = 1000
GRID = N // NB


def _first_body(p_ref, x_ref, w_ref, g_ref, u_ref):
    p = p_ref[...]
    deg = p[0, :, :1] + p[1, :, :1] + 1.0
    u = lax.rsqrt(deg)
    u_ref[...] = jnp.broadcast_to(u, (NB, 16))
    h = jnp.dot(x_ref[...], w_ref[...], preferred_element_type=jnp.float32,
                precision=lax.Precision.HIGHEST)
    g_ref[...] = h * u


def _mid_body(s_ref, g_ref, u_ref, b_ref, w_ref, o_ref):
    sv = s_ref[...]
    u = u_ref[...][:, :1]
    t = (sv[0] + sv[1] + g_ref[...]) * u + b_ref[...]
    z = jnp.maximum(t, 0.0)
    o_ref[...] = jnp.dot(z, w_ref[...], preferred_element_type=jnp.float32,
                         precision=lax.Precision.HIGHEST) * u


def _last_body(s_ref, g_ref, u_ref, b_ref, o_ref):
    sv = s_ref[...]
    u = u_ref[...][:, :1]
    o_ref[...] = (sv[0] + sv[1] + g_ref[...]) * u + b_ref[...]


_spec_p = pl.BlockSpec((2, NB, D), lambda i: (0, i, 0))
_spec_x = pl.BlockSpec((NB, D), lambda i: (i, 0))
_spec_w = pl.BlockSpec((D, D), lambda i: (0, 0))
_spec_s = pl.BlockSpec((2, NB, D), lambda i: (0, i, 0))
_spec_u = pl.BlockSpec((NB, 16), lambda i: (i, 0))
_spec_b = pl.BlockSpec((1, D), lambda i: (0, 0))

_first_tc = pl.pallas_call(
    _first_body,
    grid=(GRID,),
    in_specs=[_spec_p, _spec_x, _spec_w],
    out_specs=[_spec_x, _spec_u],
    out_shape=[jax.ShapeDtypeStruct((N, D), jnp.float32),
               jax.ShapeDtypeStruct((N, 16), jnp.float32)],
)

_mid_tc = pl.pallas_call(
    _mid_body,
    grid=(GRID,),
    in_specs=[_spec_s, _spec_x, _spec_u, _spec_b, _spec_w],
    out_specs=_spec_x,
    out_shape=jax.ShapeDtypeStruct((N, D), jnp.float32),
)

_last_tc = pl.pallas_call(
    _last_body,
    grid=(GRID,),
    in_specs=[_spec_s, _spec_x, _spec_u, _spec_b],
    out_specs=_spec_x,
    out_shape=jax.ShapeDtypeStruct((N, D), jnp.float32),
)


@jax.jit
def kernel(x, adj_t, W1, b1, W2, b2, W3, b3):
    adj = adj_t.astype(jnp.int32)
    src = jnp.concatenate([adj[0], jnp.zeros((E_PAD - E,), jnp.int32)])
    pad_dst = DUMMY + jnp.arange(E_PAD - E, dtype=jnp.int32) % (NACC - N)
    dst = jnp.concatenate([adj[1], pad_dst])
    dst2d = dst.reshape(-1, C)
    onesCD = jnp.ones((C, D), jnp.float32)
    zerosAD = jnp.zeros((NACC, D), jnp.float32)

    p = _deg_kernel(dst2d, onesCD, zerosAD)
    g1, u16 = _first_tc(p, x, W1)
    s1 = _spmm_kernel(g1, src, dst, zerosAD)
    g2 = _mid_tc(s1, g1, u16, b1.reshape(1, D), W2)
    s2 = _spmm_kernel(g2, src, dst, zerosAD)
    g3 = _mid_tc(s2, g2, u16, b2.reshape(1, D), W3)
    s3 = _spmm_kernel(g3, src, dst, zerosAD)
    out = _last_tc(s3, g3, u16, b3.reshape(1, D))
    return out


# speedup vs baseline: 2.0379x; 1.9144x over previous
import functools

import jax
import jax.numpy as jnp
from jax import lax
from jax.experimental import pallas as pl
from jax.experimental.pallas import tpu as pltpu
from jax.experimental.pallas import tpu_sc as plsc

N = 10000
E = 320000
D = 128

NC = 2
NS = 16
NW = NC * NS
C = 128
NCH = E // C
BASE_CH = NCH // NW
EXTRA = NCH - BASE_CH * NW
CH_PER_TILE = 80
E_PAD = NW * CH_PER_TILE * C
NACC = 10240
DUMMY = N
ACC_PER_TILE = NACC // NS
ROWS_PER_TILE = 624
TAIL_R0 = ROWS_PER_TILE * NS
TAIL_ROWS = N - TAIL_R0


def _copy_rows(copy_fn, s):
    copy_fn(s * ROWS_PER_TILE, ROWS_PER_TILE)

    @pl.when(s == NS - 1)
    def _():
        copy_fn(TAIL_R0, TAIL_ROWS)

_mesh = plsc.VectorSubcoreMesh(core_axis_name="c", subcore_axis_name="s")


def _init_accum(zeros_hbm, accum, s):
    r0 = s * ACC_PER_TILE
    pltpu.sync_copy(zeros_hbm.at[pl.ds(r0, ACC_PER_TILE)],
                    accum.at[pl.ds(r0, ACC_PER_TILE)])


def _writeback(accum, out_hbm, c, s):
    _copy_rows(lambda r0, nr: pltpu.sync_copy(
        accum.at[pl.ds(r0, nr)], out_hbm.at[c, pl.ds(r0, nr)]), s)


@functools.partial(
    pl.kernel,
    mesh=_mesh,
    out_type=jax.ShapeDtypeStruct((NC, N, D), jnp.float32),
    scratch_types=[
        pltpu.VMEM((CH_PER_TILE, C), jnp.int32),
        pltpu.VMEM((C, D), jnp.float32),
        pltpu.VMEM_SHARED((NACC, D), jnp.float32),
    ],
)
def _deg_kernel(dst_hbm, ones_hbm, zeros_hbm, out_hbm, dst_all, ones_v, accum):
    c = lax.axis_index("c")
    s = lax.axis_index("s")
    wid = s * NC + c
    _init_accum(zeros_hbm, accum, s)
    pltpu.sync_copy(dst_hbm.at[pl.ds(wid * CH_PER_TILE, CH_PER_TILE)], dst_all)
    pltpu.sync_copy(ones_hbm, ones_v)
    plsc.subcore_barrier()

    def body(j, carry):
        pltpu.sync_copy(ones_v, accum.at[dst_all.at[j]], add=True)
        return carry

    lax.fori_loop(0, CH_PER_TILE, body, 0)
    plsc.subcore_barrier()
    _writeback(accum, out_hbm, c, s)


@functools.partial(
    pl.kernel,
    mesh=_mesh,
    out_type=jax.ShapeDtypeStruct((NC, N, D), jnp.float32),
    scratch_types=[
        pltpu.VMEM((C,), jnp.int32),
        pltpu.VMEM((C,), jnp.int32),
        pltpu.VMEM((C, D), jnp.float32),
        pltpu.VMEM_SHARED((N, D), jnp.float32),
        pltpu.SemaphoreType.DMA,
    ],
)
def _spmm_kernel(g_hbm, src_hbm, dst_hbm, zeros_hbm, out_hbm,
                 src_v, dst_v, rows_v, accum, sem):
    c = lax.axis_index("c")
    s = lax.axis_index("s")
    wid = s * NC + c
    _copy_rows(lambda r0, nr: pltpu.sync_copy(
        zeros_hbm.at[pl.ds(r0, nr)], accum.at[pl.ds(r0, nr)]), s)
    plsc.subcore_barrier()

    n_my = jnp.where(wid < EXTRA, BASE_CH + 1, BASE_CH)

    def body(k, carry):
        ch = wid + k * NW
        pltpu.sync_copy(src_hbm.at[pl.ds(ch * C, C)], src_v)
        pltpu.sync_copy(dst_hbm.at[pl.ds(ch * C, C)], dst_v)
        pltpu.async_copy(g_hbm.at[src_v], rows_v, sem).wait()
        pltpu.sync_copy(rows_v, accum.at[dst_v], add=True)
        return carry

    lax.fori_loop(0, n_my, body, 0)
    plsc.subcore_barrier()
    _writeback(accum, out_hbm, c, s)


NB = 1000
GRID = N // NB


def _first_body(p_ref, x_ref, w_ref, g_ref, u_ref):
    p = p_ref[...]
    deg = p[0, :, :1] + p[1, :, :1] + 1.0
    u = lax.rsqrt(deg)
    u_ref[...] = jnp.broadcast_to(u, (NB, 16))
    h = jnp.dot(x_ref[...], w_ref[...], preferred_element_type=jnp.float32,
                precision=lax.Precision.HIGHEST)
    g_ref[...] = h * u


def _mid_body(s_ref, g_ref, u_ref, b_ref, w_ref, o_ref):
    sv = s_ref[...]
    u = u_ref[...][:, :1]
    t = (sv[0] + sv[1] + g_ref[...]) * u + b_ref[...]
    z = jnp.maximum(t, 0.0)
    o_ref[...] = jnp.dot(z, w_ref[...], preferred_element_type=jnp.float32,
                         precision=lax.Precision.HIGHEST) * u


def _last_body(s_ref, g_ref, u_ref, b_ref, o_ref):
    sv = s_ref[...]
    u = u_ref[...][:, :1]
    o_ref[...] = (sv[0] + sv[1] + g_ref[...]) * u + b_ref[...]


_spec_p = pl.BlockSpec((2, NB, D), lambda i: (0, i, 0))
_spec_x = pl.BlockSpec((NB, D), lambda i: (i, 0))
_spec_w = pl.BlockSpec((D, D), lambda i: (0, 0))
_spec_s = pl.BlockSpec((2, NB, D), lambda i: (0, i, 0))
_spec_u = pl.BlockSpec((NB, 16), lambda i: (i, 0))
_spec_b = pl.BlockSpec((1, D), lambda i: (0, 0))

_first_tc = pl.pallas_call(
    _first_body,
    grid=(GRID,),
    in_specs=[_spec_p, _spec_x, _spec_w],
    out_specs=[_spec_x, _spec_u],
    out_shape=[jax.ShapeDtypeStruct((N, D), jnp.float32),
               jax.ShapeDtypeStruct((N, 16), jnp.float32)],
)

_mid_tc = pl.pallas_call(
    _mid_body,
    grid=(GRID,),
    in_specs=[_spec_s, _spec_x, _spec_u, _spec_b, _spec_w],
    out_specs=_spec_x,
    out_shape=jax.ShapeDtypeStruct((N, D), jnp.float32),
)

_last_tc = pl.pallas_call(
    _last_body,
    grid=(GRID,),
    in_specs=[_spec_s, _spec_x, _spec_u, _spec_b],
    out_specs=_spec_x,
    out_shape=jax.ShapeDtypeStruct((N, D), jnp.float32),
)


@jax.jit
def kernel(x, adj_t, W1, b1, W2, b2, W3, b3):
    adj = adj_t.astype(jnp.int32)
    src = jnp.concatenate([adj[0], jnp.zeros((E_PAD - E,), jnp.int32)])
    pad_dst = DUMMY + jnp.arange(E_PAD - E, dtype=jnp.int32) % (NACC - N)
    dst = jnp.concatenate([adj[1], pad_dst])
    dst2d = dst.reshape(-1, C)
    onesCD = jnp.ones((C, D), jnp.float32)
    zerosAD = jnp.zeros((NACC, D), jnp.float32)
    zerosND = jnp.zeros((N, D), jnp.float32)
    srcE = adj[0]
    dstE = adj[1]

    p = _deg_kernel(dst2d, onesCD, zerosAD)
    g1, u16 = _first_tc(p, x, W1)
    s1 = _spmm_kernel(g1, srcE, dstE, zerosND)
    g2 = _mid_tc(s1, g1, u16, b1.reshape(1, D), W2)
    s2 = _spmm_kernel(g2, srcE, dstE, zerosND)
    g3 = _mid_tc(s2, g2, u16, b2.reshape(1, D), W3)
    s3 = _spmm_kernel(g3, srcE, dstE, zerosND)
    out = _last_tc(s3, g3, u16, b3.reshape(1, D))
    return out


# speedup vs baseline: 2.8840x; 1.4152x over previous
import functools

import jax
import jax.numpy as jnp
from jax import lax
from jax.experimental import pallas as pl
from jax.experimental.pallas import tpu as pltpu
from jax.experimental.pallas import tpu_sc as plsc

N = 10000
E = 320000
D = 128

NC = 2
NS = 16
NW = NC * NS
C = 128
NCH = E // C
BASE_CH = NCH // NW
EXTRA = NCH - BASE_CH * NW
CH_PER_TILE = 80
E_PAD = NW * CH_PER_TILE * C
NACC = 10240
DUMMY = N
ACC_PER_TILE = NACC // NS
ROWS_PER_TILE = 624
TAIL_R0 = ROWS_PER_TILE * NS
TAIL_ROWS = N - TAIL_R0


def _copy_rows(copy_fn, s):
    copy_fn(s * ROWS_PER_TILE, ROWS_PER_TILE)

    @pl.when(s == NS - 1)
    def _():
        copy_fn(TAIL_R0, TAIL_ROWS)

_mesh = plsc.VectorSubcoreMesh(core_axis_name="c", subcore_axis_name="s")


def _init_accum(zeros_hbm, accum, s):
    r0 = s * ACC_PER_TILE
    pltpu.sync_copy(zeros_hbm.at[pl.ds(r0, ACC_PER_TILE)],
                    accum.at[pl.ds(r0, ACC_PER_TILE)])


def _writeback(accum, out_hbm, c, s):
    _copy_rows(lambda r0, nr: pltpu.sync_copy(
        accum.at[pl.ds(r0, nr)], out_hbm.at[c, pl.ds(r0, nr)]), s)


@functools.partial(
    pl.kernel,
    mesh=_mesh,
    out_type=jax.ShapeDtypeStruct((NC, N, D), jnp.float32),
    scratch_types=[
        pltpu.VMEM((CH_PER_TILE, C), jnp.int32),
        pltpu.VMEM((C, D), jnp.float32),
        pltpu.VMEM_SHARED((NACC, D), jnp.float32),
    ],
)
def _deg_kernel(dst_hbm, ones_hbm, zeros_hbm, out_hbm, dst_all, ones_v, accum):
    c = lax.axis_index("c")
    s = lax.axis_index("s")
    wid = s * NC + c
    _init_accum(zeros_hbm, accum, s)
    pltpu.sync_copy(dst_hbm.at[pl.ds(wid * CH_PER_TILE, CH_PER_TILE)], dst_all)
    pltpu.sync_copy(ones_hbm, ones_v)
    plsc.subcore_barrier()

    def body(j, carry):
        pltpu.sync_copy(ones_v, accum.at[dst_all.at[j]], add=True)
        return carry

    lax.fori_loop(0, CH_PER_TILE, body, 0)
    plsc.subcore_barrier()
    _writeback(accum, out_hbm, c, s)


@functools.partial(
    pl.kernel,
    mesh=_mesh,
    out_type=jax.ShapeDtypeStruct((NC, N, D), jnp.float32),
    scratch_types=[
        pltpu.VMEM((C,), jnp.int32),
        pltpu.VMEM((C,), jnp.int32),
        pltpu.VMEM((C,), jnp.int32),
        pltpu.VMEM((C, D), jnp.float32),
        pltpu.VMEM((C, D), jnp.float32),
        pltpu.VMEM_SHARED((NACC, D), jnp.float32),
        pltpu.SemaphoreType.DMA,
        pltpu.SemaphoreType.DMA,
    ],
)
def _spmm_kernel(g_hbm, src_hbm, dst_hbm, zeros_hbm, out_hbm,
                 dbuf, sbuf0, sbuf1, rows_a, rows_b, accum, sa, sb):
    c = lax.axis_index("c")
    s = lax.axis_index("s")
    wid = s * NC + c
    _init_accum(zeros_hbm, accum, s)
    plsc.subcore_barrier()

    def src_slice(j):
        return src_hbm.at[pl.ds((j * NW + wid) * C, C)]

    def dst_slice(j):
        return dst_hbm.at[pl.ds((j * NW + wid) * C, C)]

    W = 10
    sbufs = (sbuf0, sbuf1)
    rows = (rows_a, rows_b)
    sems = (sa, sb)

    def gather(sl, buf, sem):
        return pltpu.async_copy(g_hbm.at[sl], buf, sem, priority=1)

    def window(w, carry):
        j0 = w * W
        pltpu.sync_copy(src_slice(j0), sbufs[0])
        g0 = gather(sbufs[0], rows[0], sems[0])
        pltpu.sync_copy(src_slice(j0 + 1), sbufs[1])
        g1 = gather(sbufs[1], rows[1], sems[1])
        gs = [g0, g1]
        for t in range(W):
            b = t % 2
            gs[b].wait()
            pltpu.sync_copy(dst_slice(j0 + t), dbuf)
            pltpu.sync_copy(rows[b], accum.at[dbuf], add=True)
            if t + 2 < W:
                pltpu.sync_copy(src_slice(j0 + t + 2), sbufs[b])
                gs[b] = gather(sbufs[b], rows[b], sems[b])
        return carry

    lax.fori_loop(0, CH_PER_TILE // W, window, 0)
    plsc.subcore_barrier()
    _writeback(accum, out_hbm, c, s)


NB = 1000
GRID = N // NB


def _first_body(p_ref, x_ref, w_ref, g_ref, u_ref):
    p = p_ref[...]
    deg = p[0, :, :1] + p[1, :, :1] + 1.0
    u = lax.rsqrt(deg)
    u_ref[...] = jnp.broadcast_to(u, (NB, 16))
    h = jnp.dot(x_ref[...], w_ref[...], preferred_element_type=jnp.float32,
                precision=lax.Precision.HIGHEST)
    g_ref[...] = h * u


def _mid_body(s_ref, g_ref, u_ref, b_ref, w_ref, o_ref):
    sv = s_ref[...]
    u = u_ref[...][:, :1]
    t = (sv[0] + sv[1] + g_ref[...]) * u + b_ref[...]
    z = jnp.maximum(t, 0.0)
    o_ref[...] = jnp.dot(z, w_ref[...], preferred_element_type=jnp.float32,
                         precision=lax.Precision.HIGHEST) * u


def _last_body(s_ref, g_ref, u_ref, b_ref, o_ref):
    sv = s_ref[...]
    u = u_ref[...][:, :1]
    o_ref[...] = (sv[0] + sv[1] + g_ref[...]) * u + b_ref[...]


_spec_p = pl.BlockSpec((2, NB, D), lambda i: (0, i, 0))
_spec_x = pl.BlockSpec((NB, D), lambda i: (i, 0))
_spec_w = pl.BlockSpec((D, D), lambda i: (0, 0))
_spec_s = pl.BlockSpec((2, NB, D), lambda i: (0, i, 0))
_spec_u = pl.BlockSpec((NB, 16), lambda i: (i, 0))
_spec_b = pl.BlockSpec((1, D), lambda i: (0, 0))

_first_tc = pl.pallas_call(
    _first_body,
    grid=(GRID,),
    in_specs=[_spec_p, _spec_x, _spec_w],
    out_specs=[_spec_x, _spec_u],
    out_shape=[jax.ShapeDtypeStruct((N, D), jnp.float32),
               jax.ShapeDtypeStruct((N, 16), jnp.float32)],
)

_mid_tc = pl.pallas_call(
    _mid_body,
    grid=(GRID,),
    in_specs=[_spec_s, _spec_x, _spec_u, _spec_b, _spec_w],
    out_specs=_spec_x,
    out_shape=jax.ShapeDtypeStruct((N, D), jnp.float32),
)

_last_tc = pl.pallas_call(
    _last_body,
    grid=(GRID,),
    in_specs=[_spec_s, _spec_x, _spec_u, _spec_b],
    out_specs=_spec_x,
    out_shape=jax.ShapeDtypeStruct((N, D), jnp.float32),
)


@jax.jit
def kernel(x, adj_t, W1, b1, W2, b2, W3, b3):
    adj = adj_t.astype(jnp.int32)
    pad_iota = jnp.arange(E_PAD - E, dtype=jnp.int32)
    src = jnp.concatenate([adj[0], pad_iota % N])
    dst = jnp.concatenate([adj[1], DUMMY + pad_iota % (NACC - N)])
    dst2d = dst.reshape(-1, C)
    onesCD = jnp.ones((C, D), jnp.float32)
    zerosAD = jnp.zeros((NACC, D), jnp.float32)

    p = _deg_kernel(dst2d, onesCD, zerosAD)
    g1, u16 = _first_tc(p, x, W1)
    s1 = _spmm_kernel(g1, src, dst, zerosAD)
    g2 = _mid_tc(s1, g1, u16, b1.reshape(1, D), W2)
    s2 = _spmm_kernel(g2, src, dst, zerosAD)
    g3 = _mid_tc(s2, g2, u16, b2.reshape(1, D), W3)
    s3 = _spmm_kernel(g3, src, dst, zerosAD)
    out = _last_tc(s3, g3, u16, b3.reshape(1, D))
    return out
